# baseline (speedup 1.0000x reference)
import functools

import jax
import jax.numpy as jnp
from jax import lax
from jax.experimental import pallas as pl
from jax.experimental.pallas import tpu as pltpu
from jax.experimental.pallas import tpu_sc as plsc

VOCAB = 100000
E = 128
H = 128
B = 1024
L = 200
NG = 4 * H

_NC = 2
_NS = 16
_NW = _NC * _NS
_ROWS = B * L
_PER_W = _ROWS // _NW
_CH = 128
_NCH = _PER_W // _CH


@functools.cache
def _make_embed_gather():
    @functools.partial(
        pl.kernel,
        mesh=plsc.VectorSubcoreMesh(core_axis_name="c", subcore_axis_name="s"),
        out_type=jax.ShapeDtypeStruct((_ROWS, E), jnp.float32),
        scratch_types=[
            pltpu.VMEM((_CH,), jnp.int32),
            pltpu.VMEM((_CH, E), jnp.float32),
            pltpu.SemaphoreType.DMA,
        ],
    )
    def _embed_gather(emb_hbm, idx_hbm, out_hbm, idx_v, rows_v, sem):
        wid = lax.axis_index("s") * _NC + lax.axis_index("c")
        base = wid * _PER_W

        def body(i, carry):
            off = base + i * _CH
            pltpu.sync_copy(idx_hbm.at[pl.ds(off, _CH)], idx_v)
            pltpu.async_copy(emb_hbm.at[idx_v], rows_v, sem).wait()
            pltpu.sync_copy(rows_v, out_hbm.at[pl.ds(off, _CH)])
            return carry

        lax.fori_loop(0, _NCH, body, 0)

    return _embed_gather


def _lstm_body(xf_ref, xb_ref, wf_ref, wr_ref,
               bf_ref, br_ref, wfc_ref, bfc_ref, out_ref,
               hf, cf, hb, cb):
    t = pl.program_id(0)

    @pl.when(t == 0)
    def _init():
        hf[...] = jnp.zeros_like(hf)
        cf[...] = jnp.zeros_like(cf)
        hb[...] = jnp.zeros_like(hb)
        cb[...] = jnp.zeros_like(cb)

    def step(x, h, c, w, b):
        xh = jnp.concatenate([x, h], axis=-1)
        gates = jnp.dot(xh, w, preferred_element_type=jnp.float32) + b
        t_i = jnp.tanh(gates[:, 0 * H:1 * H])
        f_g = jax.nn.sigmoid(gates[:, 1 * H:2 * H])
        g_t = jnp.tanh(gates[:, 2 * H:3 * H])
        t_o = jnp.tanh(gates[:, 3 * H:4 * H])
        c_new = f_g * c + 0.5 * (t_i * g_t + g_t)
        ct = jnp.tanh(c_new)
        h_new = 0.5 * (t_o * ct + ct)
        return h_new, c_new

    hf_new, cf_new = step(xf_ref[0], hf[...], cf[...],
                          wf_ref[...], bf_ref[...])
    hb_new, cb_new = step(xb_ref[0], hb[...], cb[...],
                          wr_ref[...], br_ref[...])
    hf[...] = hf_new
    cf[...] = cf_new
    hb[...] = hb_new
    cb[...] = cb_new

    @pl.when(t == L - 1)
    def _finish():
        wfc = wfc_ref[...]
        logits = jnp.sum(hf_new * wfc[:, :H] + hb_new * wfc[:, H:],
                         axis=1, keepdims=True) + bfc_ref[0, 0]
        out_ref[...] = jnp.broadcast_to(logits, (B, E))


_lstm_call = pl.pallas_call(
    _lstm_body,
    grid=(L,),
    in_specs=[
        pl.BlockSpec((1, B, E), lambda t: (t, 0, 0)),
        pl.BlockSpec((1, B, E), lambda t: (L - 1 - t, 0, 0)),
        pl.BlockSpec((E + H, NG), lambda t: (0, 0)),
        pl.BlockSpec((E + H, NG), lambda t: (0, 0)),
        pl.BlockSpec((1, NG), lambda t: (0, 0)),
        pl.BlockSpec((1, NG), lambda t: (0, 0)),
        pl.BlockSpec((1, 2 * H), lambda t: (0, 0)),
        pl.BlockSpec(memory_space=pltpu.SMEM),
    ],
    out_specs=pl.BlockSpec((B, E), lambda t: (0, 0)),
    out_shape=jax.ShapeDtypeStruct((B, E), jnp.float32),
    scratch_shapes=[pltpu.VMEM((B, H), jnp.float32)] * 4,
    compiler_params=pltpu.CompilerParams(
        dimension_semantics=("arbitrary",)),
)


def _prep_w(Wih, Whh, bih, bhh):
    w = jnp.concatenate([Wih.T, Whh.T], axis=0)
    b = (bih + bhh).reshape(1, NG)
    scale = jnp.concatenate([
        jnp.full((H,), 0.5, jnp.float32),
        jnp.ones((H,), jnp.float32),
        jnp.ones((H,), jnp.float32),
        jnp.full((H,), 0.5, jnp.float32),
    ])
    return w * scale, b * scale


def kernel(x, emb, Wih_f, Whh_f, bih_f, bhh_f,
           Wih_r, Whh_r, bih_r, bhh_r, W_fc, b_fc):
    idx = jnp.transpose(x).reshape(-1).astype(jnp.int32)
    xs = _make_embed_gather()(emb, idx).reshape(L, B, E)
    wf, bf = _prep_w(Wih_f, Whh_f, bih_f, bhh_f)
    wr, br = _prep_w(Wih_r, Whh_r, bih_r, bhh_r)
    out = _lstm_call(xs, xs, wf, wr, bf, br, W_fc, b_fc.reshape(1, 1))
    return out[:, 0]


# speedup vs baseline: 1.1183x; 1.1183x over previous
import functools

import jax
import jax.numpy as jnp
from jax import lax
from jax.experimental import pallas as pl
from jax.experimental.pallas import tpu as pltpu
from jax.experimental.pallas import tpu_sc as plsc

VOCAB = 100000
E = 128
H = 128
B = 1024
L = 200
NG = 4 * H

_NC = 2
_NS = 16
_NW = _NC * _NS
_ROWS = B * L
_PER_W = _ROWS // _NW
_CH = 128
_NCH = _PER_W // _CH


@functools.cache
def _make_embed_gather():
    @functools.partial(
        pl.kernel,
        mesh=plsc.VectorSubcoreMesh(core_axis_name="c", subcore_axis_name="s"),
        out_type=jax.ShapeDtypeStruct((_ROWS, E), jnp.float32),
        scratch_types=[
            pltpu.VMEM((_PER_W,), jnp.int32),
            pltpu.VMEM((_CH, E), jnp.float32),
            pltpu.VMEM((_CH, E), jnp.float32),
            pltpu.SemaphoreType.DMA,
            pltpu.SemaphoreType.DMA,
        ],
    )
    def _embed_gather(emb_hbm, idx_hbm, out_hbm, idx_v, rows0, rows1, s0, s1):
        wid = lax.axis_index("s") * _NC + lax.axis_index("c")
        base = wid * _PER_W
        pltpu.sync_copy(idx_hbm.at[pl.ds(base, _PER_W)], idx_v)

        def gather(i, rows, sem):
            pltpu.async_copy(emb_hbm.at[idx_v.at[pl.ds(i * _CH, _CH)]],
                             rows, sem)

        def drain(rows, sem):
            pltpu.make_async_copy(emb_hbm.at[pl.ds(0, _CH)], rows, sem).wait()

        gather(0, rows0, s0)

        def body(j, carry):
            i = 2 * j
            drain(rows0, s0)
            gather(i + 1, rows1, s1)
            pltpu.sync_copy(rows0, out_hbm.at[pl.ds(base + i * _CH, _CH)])
            drain(rows1, s1)

            @pl.when(i + 2 < _NCH)
            def _():
                gather(i + 2, rows0, s0)

            pltpu.sync_copy(rows1, out_hbm.at[pl.ds(base + (i + 1) * _CH, _CH)])
            return carry

        lax.fori_loop(0, _NCH // 2, body, 0)

    return _embed_gather


def _lstm_body(xf_ref, xb_ref, wf_ref, wr_ref,
               bf_ref, br_ref, wfc_ref, bfc_ref, out_ref,
               hf, cf, hb, cb):
    t = pl.program_id(0)

    @pl.when(t == 0)
    def _init():
        hf[...] = jnp.zeros_like(hf)
        cf[...] = jnp.zeros_like(cf)
        hb[...] = jnp.zeros_like(hb)
        cb[...] = jnp.zeros_like(cb)

    def step(x, h, c, w, b):
        xh = jnp.concatenate([x, h], axis=-1)
        gates = jnp.dot(xh, w, preferred_element_type=jnp.float32) + b
        t_i = jnp.tanh(gates[:, 0 * H:1 * H])
        f_g = jax.nn.sigmoid(gates[:, 1 * H:2 * H])
        g_t = jnp.tanh(gates[:, 2 * H:3 * H])
        t_o = jnp.tanh(gates[:, 3 * H:4 * H])
        c_new = f_g * c + 0.5 * (t_i * g_t + g_t)
        ct = jnp.tanh(c_new)
        h_new = 0.5 * (t_o * ct + ct)
        return h_new, c_new

    hf_new, cf_new = step(xf_ref[0], hf[...], cf[...],
                          wf_ref[...], bf_ref[...])
    hb_new, cb_new = step(xb_ref[0], hb[...], cb[...],
                          wr_ref[...], br_ref[...])
    hf[...] = hf_new
    cf[...] = cf_new
    hb[...] = hb_new
    cb[...] = cb_new

    @pl.when(t == L - 1)
    def _finish():
        wfc = wfc_ref[...]
        logits = jnp.sum(hf_new * wfc[:, :H] + hb_new * wfc[:, H:],
                         axis=1, keepdims=True) + bfc_ref[0, 0]
        out_ref[...] = jnp.broadcast_to(logits, (B, E))


_lstm_call = pl.pallas_call(
    _lstm_body,
    grid=(L,),
    in_specs=[
        pl.BlockSpec((1, B, E), lambda t: (t, 0, 0)),
        pl.BlockSpec((1, B, E), lambda t: (L - 1 - t, 0, 0)),
        pl.BlockSpec((E + H, NG), lambda t: (0, 0)),
        pl.BlockSpec((E + H, NG), lambda t: (0, 0)),
        pl.BlockSpec((1, NG), lambda t: (0, 0)),
        pl.BlockSpec((1, NG), lambda t: (0, 0)),
        pl.BlockSpec((1, 2 * H), lambda t: (0, 0)),
        pl.BlockSpec(memory_space=pltpu.SMEM),
    ],
    out_specs=pl.BlockSpec((B, E), lambda t: (0, 0)),
    out_shape=jax.ShapeDtypeStruct((B, E), jnp.float32),
    scratch_shapes=[pltpu.VMEM((B, H), jnp.float32)] * 4,
    compiler_params=pltpu.CompilerParams(
        dimension_semantics=("arbitrary",)),
)


def _prep_w(Wih, Whh, bih, bhh):
    w = jnp.concatenate([Wih.T, Whh.T], axis=0)
    b = (bih + bhh).reshape(1, NG)
    scale = jnp.concatenate([
        jnp.full((H,), 0.5, jnp.float32),
        jnp.ones((H,), jnp.float32),
        jnp.ones((H,), jnp.float32),
        jnp.full((H,), 0.5, jnp.float32),
    ])
    return w * scale, b * scale


def kernel(x, emb, Wih_f, Whh_f, bih_f, bhh_f,
           Wih_r, Whh_r, bih_r, bhh_r, W_fc, b_fc):
    idx = jnp.transpose(x).reshape(-1).astype(jnp.int32)
    xs = _make_embed_gather()(emb, idx).reshape(L, B, E)
    wf, bf = _prep_w(Wih_f, Whh_f, bih_f, bhh_f)
    wr, br = _prep_w(Wih_r, Whh_r, bih_r, bhh_r)
    out = _lstm_call(xs, xs, wf, wr, bf, br, W_fc, b_fc.reshape(1, 1))
    return out[:, 0]
